# baseline (speedup 1.0000x reference)
import jax
import jax.numpy as jnp
from jax import lax
from jax.experimental import pallas as pl
from jax.experimental.pallas import tpu as pltpu
from jax.experimental.pallas import tpu_sc as plsc

N = 100000
H = 128
S = 512
NC = 2
NS = 16
NW = NC * NS
CHUNK = 112
BASE = 3136
LAST = N - (NW - 1) * BASE
NCH = BASE // CHUNK
NCH_LAST = LAST // CHUNK
TAIL = LAST - NCH_LAST * CHUNK
ROWS_PER_TILE = S // NS
GCH = NW * NCH


def _pool_body(h_hbm, b_hbm, b2_hbm, iden_hbm, z128_hbm, z512_hbm,
               part_out, cnt_out,
               acc_sh, acc_v, rows0_v, rows1_v, idx2_v, iden_v, cnt_v,
               tidx_v, sem0, sem1):
    c = lax.axis_index("c")
    s = lax.axis_index("s")
    wid = c * NS + s
    base = wid * BASE

    pltpu.sync_copy(z128_hbm.at[pl.ds(s * ROWS_PER_TILE, ROWS_PER_TILE)],
                    acc_sh.at[pl.ds(s * ROWS_PER_TILE, ROWS_PER_TILE)])
    pltpu.sync_copy(z512_hbm, cnt_v)
    pltpu.sync_copy(b2_hbm.at[wid], idx2_v)
    pltpu.sync_copy(iden_hbm, iden_v)

    zero16 = jnp.zeros((16,), jnp.float32)

    def _zrow(r, _):
        for k in range(H // 16):
            acc_v[r, pl.ds(16 * k, 16)] = zero16
        return _

    lax.fori_loop(0, S, _zrow, 0)

    plsc.subcore_barrier()

    nch2 = jnp.where(wid == NW - 1, NCH_LAST // 2, NCH // 2)
    ones16 = jnp.full((16,), 1.0, jnp.float32)
    lane = lax.iota(jnp.int32, 16)
    cols = [lane + 16 * k for k in range(H // 16)]

    def _off(i):
        return base + i * CHUNK

    def _accum(i, rows_v, ngroups):
        def _group(g, _):
            ids16 = idx2_v[i, pl.ds(16 * g, 16)]
            plsc.addupdate_scatter(cnt_v, [ids16], ones16)
            for r in range(16):
                seg = plsc.load_gather(
                    idx2_v, [jnp.full((16,), i, jnp.int32),
                             jnp.full((16,), 16 * g + r, jnp.int32)])
                for k in range(H // 16):
                    x = rows_v[16 * g + r, pl.ds(16 * k, 16)]
                    plsc.addupdate_scatter(acc_v, [seg, cols[k]], x)
            return _

        lax.fori_loop(0, ngroups, _group, 0)

    pltpu.async_copy(h_hbm.at[pl.ds(base, CHUNK)], rows0_v, sem0)

    def _pair(j, _):
        i0 = 2 * j
        i1 = 2 * j + 1
        pltpu.async_copy(h_hbm.at[pl.ds(_off(i1), CHUNK)], rows1_v, sem1)
        pltpu.make_async_copy(h_hbm.at[pl.ds(_off(i0), CHUNK)], rows0_v,
                              sem0).wait()
        _accum(i0, rows0_v, CHUNK // 16)
        off2 = jnp.minimum(_off(i0 + 2), N - CHUNK)
        pltpu.async_copy(h_hbm.at[pl.ds(off2, CHUNK)], rows0_v, sem0)
        pltpu.make_async_copy(h_hbm.at[pl.ds(_off(i1), CHUNK)], rows1_v,
                              sem1).wait()
        _accum(i1, rows1_v, CHUNK // 16)
        return _

    lax.fori_loop(0, nch2, _pair, 0)

    pltpu.make_async_copy(h_hbm.at[pl.ds(0, CHUNK)], rows0_v, sem0).wait()

    @pl.when(wid == NW - 1)
    def _tail():
        off = base + NCH_LAST * CHUNK
        pltpu.sync_copy(b_hbm.at[pl.ds(off, TAIL)], tidx_v)
        pltpu.sync_copy(h_hbm.at[pl.ds(off, TAIL)],
                        rows0_v.at[pl.ds(0, TAIL)])

        def _tgroup(g, _):
            ids16 = tidx_v[pl.ds(16 * g, 16)]
            plsc.addupdate_scatter(cnt_v, [ids16], ones16)
            for r in range(16):
                seg = plsc.load_gather(
                    tidx_v, [jnp.full((16,), 16 * g + r, jnp.int32)])
                for k in range(H // 16):
                    x = rows0_v[16 * g + r, pl.ds(16 * k, 16)]
                    plsc.addupdate_scatter(acc_v, [seg, cols[k]], x)
            return _

        lax.fori_loop(0, TAIL // 16, _tgroup, 0)

    for m in range(S // 128):
        pltpu.sync_copy(acc_v.at[pl.ds(128 * m, 128)],
                        acc_sh.at[iden_v.at[m]], add=True)

    plsc.subcore_barrier()

    r0 = s * ROWS_PER_TILE
    pltpu.sync_copy(acc_sh.at[pl.ds(r0, ROWS_PER_TILE)],
                    part_out.at[c, pl.ds(r0, ROWS_PER_TILE)])
    pltpu.sync_copy(cnt_v, cnt_out.at[c, s])


@jax.jit
def _sc_pool(h, b32, b2d, iden, z128, z512):
    mesh = plsc.VectorSubcoreMesh(core_axis_name="c", subcore_axis_name="s")
    f = pl.kernel(
        _pool_body,
        out_type=(
            jax.ShapeDtypeStruct((NC, S, H), jnp.float32),
            jax.ShapeDtypeStruct((NC, NS, S), jnp.float32),
        ),
        mesh=mesh,
        compiler_params=pltpu.CompilerParams(needs_layout_passes=False),
        scratch_types=[
            pltpu.VMEM_SHARED((S, H), jnp.float32),
            pltpu.VMEM((S, H), jnp.float32),
            pltpu.VMEM((CHUNK, H), jnp.float32),
            pltpu.VMEM((CHUNK, H), jnp.float32),
            pltpu.VMEM((NCH, CHUNK), jnp.int32),
            pltpu.VMEM((S // 128, 128), jnp.int32),
            pltpu.VMEM((S,), jnp.float32),
            pltpu.VMEM((TAIL,), jnp.int32),
            pltpu.SemaphoreType.DMA,
            pltpu.SemaphoreType.DMA,
        ],
    )
    return f(h, b32, b2d, iden, z128, z512)


def _combine_body(p_ref, c_ref, o_ref):
    p = p_ref[0] + p_ref[1]
    cnt = jnp.sum(c_ref[...], axis=(0, 1))
    cnt = jnp.maximum(cnt, 1.0)
    o_ref[...] = p / cnt.reshape(S, 1)


@jax.jit
def _combine(part, cnt):
    return pl.pallas_call(
        _combine_body,
        out_shape=jax.ShapeDtypeStruct((S, H), jnp.float32),
    )(part, cnt)


def kernel(h, batch):
    b32 = batch.astype(jnp.int32)
    b2d = jnp.concatenate(
        [b32, jnp.zeros((GCH * CHUNK - N,), jnp.int32)]).reshape(
            NW, NCH, CHUNK)
    iden = jnp.arange(S, dtype=jnp.int32).reshape(S // 128, 128)
    z128 = jnp.zeros((S, H), jnp.float32)
    z512 = jnp.zeros((S,), jnp.float32)
    part, cnt = _sc_pool(h, b32, b2d, iden, z128, z512)
    return _combine(part, cnt)


# speedup vs baseline: 1.6263x; 1.6263x over previous
import jax
import jax.numpy as jnp
from jax import lax
from jax.experimental import pallas as pl
from jax.experimental.pallas import tpu as pltpu
from jax.experimental.pallas import tpu_sc as plsc

N = 100000
H = 128
S = 512
NC = 2
NS = 16
NW = NC * NS
CHUNK = 112
BASE = 3136
LAST = N - (NW - 1) * BASE
NCH = BASE // CHUNK
NCH_LAST = LAST // CHUNK
TAIL = LAST - NCH_LAST * CHUNK
ROWS_PER_TILE = S // NS
GCH = NW * NCH
KF = H // 16


def _pool_body(h_hbm, b_hbm, b2_hbm, iden_hbm, z128_hbm, z512_hbm,
               part_out, cnt_out,
               acc_sh, acc_v, rows0_v, rows1_v, idx2_v, iden_v, cnt_v,
               tidx_v, sem0, sem1):
    c = lax.axis_index("c")
    s = lax.axis_index("s")
    wid = c * NS + s
    base = wid * BASE

    pltpu.sync_copy(z128_hbm.at[pl.ds(s * ROWS_PER_TILE, ROWS_PER_TILE)],
                    acc_sh.at[pl.ds(s * ROWS_PER_TILE, ROWS_PER_TILE)])
    pltpu.sync_copy(z512_hbm, cnt_v)
    pltpu.sync_copy(b2_hbm.at[wid], idx2_v)
    pltpu.sync_copy(iden_hbm, iden_v)

    zero16 = jnp.zeros((16,), jnp.float32)

    def _zrow(r, _):
        for k in range(KF):
            acc_v[r, pl.ds(16 * k, 16)] = zero16
        return _

    lax.fori_loop(0, S, _zrow, 0)

    plsc.subcore_barrier()

    nch2 = jnp.where(wid == NW - 1, NCH_LAST // 2, NCH // 2)
    ones16 = jnp.full((16,), 1.0, jnp.float32)
    lane = lax.iota(jnp.int32, 16)
    cols = [lane + 16 * k for k in range(KF)]

    def _off(i):
        return base + i * CHUNK

    def _accum(i, rows_v, ngroups):
        def _group(g, _):
            ids16 = idx2_v[i, pl.ds(16 * g, 16)]
            seg0 = plsc.load_gather(
                idx2_v, [jnp.full((16,), i, jnp.int32),
                         jnp.full((16,), 16 * g, jnp.int32)])
            plsc.addupdate_scatter(cnt_v, [ids16], ones16)
            uniform = jnp.all(ids16 == seg0)

            @pl.when(uniform)
            def _fast():
                for k in range(KF):
                    acc = rows_v[16 * g, pl.ds(16 * k, 16)]
                    for r in range(1, 16):
                        acc = acc + rows_v[16 * g + r, pl.ds(16 * k, 16)]
                    plsc.addupdate_scatter(acc_v, [seg0, cols[k]], acc)

            @pl.when(jnp.logical_not(uniform))
            def _slow():
                for r in range(16):
                    segr = plsc.load_gather(
                        idx2_v, [jnp.full((16,), i, jnp.int32),
                                 jnp.full((16,), 16 * g + r, jnp.int32)])
                    for k in range(KF):
                        x = rows_v[16 * g + r, pl.ds(16 * k, 16)]
                        plsc.addupdate_scatter(acc_v, [segr, cols[k]], x)

            return _

        lax.fori_loop(0, ngroups, _group, 0)

    pltpu.async_copy(h_hbm.at[pl.ds(base, CHUNK)], rows0_v, sem0)

    def _pair(j, _):
        i0 = 2 * j
        i1 = 2 * j + 1
        pltpu.async_copy(h_hbm.at[pl.ds(_off(i1), CHUNK)], rows1_v, sem1)
        pltpu.make_async_copy(h_hbm.at[pl.ds(_off(i0), CHUNK)], rows0_v,
                              sem0).wait()
        _accum(i0, rows0_v, CHUNK // 16)
        off2 = jnp.minimum(_off(i0 + 2), N - CHUNK)
        pltpu.async_copy(h_hbm.at[pl.ds(off2, CHUNK)], rows0_v, sem0)
        pltpu.make_async_copy(h_hbm.at[pl.ds(_off(i1), CHUNK)], rows1_v,
                              sem1).wait()
        _accum(i1, rows1_v, CHUNK // 16)
        return _

    lax.fori_loop(0, nch2, _pair, 0)

    pltpu.make_async_copy(h_hbm.at[pl.ds(0, CHUNK)], rows0_v, sem0).wait()

    @pl.when(wid == NW - 1)
    def _tail():
        off = base + NCH_LAST * CHUNK
        pltpu.sync_copy(b_hbm.at[pl.ds(off, TAIL)], tidx_v)
        pltpu.sync_copy(h_hbm.at[pl.ds(off, TAIL)],
                        rows0_v.at[pl.ds(0, TAIL)])

        def _tgroup(g, _):
            ids16 = tidx_v[pl.ds(16 * g, 16)]
            plsc.addupdate_scatter(cnt_v, [ids16], ones16)
            for r in range(16):
                segr = plsc.load_gather(
                    tidx_v, [jnp.full((16,), 16 * g + r, jnp.int32)])
                for k in range(KF):
                    x = rows0_v[16 * g + r, pl.ds(16 * k, 16)]
                    plsc.addupdate_scatter(acc_v, [segr, cols[k]], x)
            return _

        lax.fori_loop(0, TAIL // 16, _tgroup, 0)

    for m in range(S // 128):
        pltpu.sync_copy(acc_v.at[pl.ds(128 * m, 128)],
                        acc_sh.at[iden_v.at[m]], add=True)

    plsc.subcore_barrier()

    r0 = s * ROWS_PER_TILE
    pltpu.sync_copy(acc_sh.at[pl.ds(r0, ROWS_PER_TILE)],
                    part_out.at[c, pl.ds(r0, ROWS_PER_TILE)])
    pltpu.sync_copy(cnt_v, cnt_out.at[c, s])


@jax.jit
def _sc_pool(h, b32, b2d, iden, z128, z512):
    mesh = plsc.VectorSubcoreMesh(core_axis_name="c", subcore_axis_name="s")
    f = pl.kernel(
        _pool_body,
        out_type=(
            jax.ShapeDtypeStruct((NC, S, H), jnp.float32),
            jax.ShapeDtypeStruct((NC, NS, S), jnp.float32),
        ),
        mesh=mesh,
        compiler_params=pltpu.CompilerParams(needs_layout_passes=False),
        scratch_types=[
            pltpu.VMEM_SHARED((S, H), jnp.float32),
            pltpu.VMEM((S, H), jnp.float32),
            pltpu.VMEM((CHUNK, H), jnp.float32),
            pltpu.VMEM((CHUNK, H), jnp.float32),
            pltpu.VMEM((NCH, CHUNK), jnp.int32),
            pltpu.VMEM((S // 128, 128), jnp.int32),
            pltpu.VMEM((S,), jnp.float32),
            pltpu.VMEM((TAIL,), jnp.int32),
            pltpu.SemaphoreType.DMA,
            pltpu.SemaphoreType.DMA,
        ],
    )
    return f(h, b32, b2d, iden, z128, z512)


def _combine_body(p_ref, c_ref, o_ref):
    p = p_ref[0] + p_ref[1]
    cnt = jnp.sum(c_ref[...], axis=(0, 1))
    cnt = jnp.maximum(cnt, 1.0)
    o_ref[...] = p / cnt.reshape(S, 1)


@jax.jit
def _combine(part, cnt):
    return pl.pallas_call(
        _combine_body,
        out_shape=jax.ShapeDtypeStruct((S, H), jnp.float32),
    )(part, cnt)


def kernel(h, batch):
    b32 = batch.astype(jnp.int32)
    b2d = jnp.concatenate(
        [b32, jnp.zeros((GCH * CHUNK - N,), jnp.int32)]).reshape(
            NW, NCH, CHUNK)
    iden = jnp.arange(S, dtype=jnp.int32).reshape(S // 128, 128)
    z128 = jnp.zeros((S, H), jnp.float32)
    z512 = jnp.zeros((S,), jnp.float32)
    part, cnt = _sc_pool(h, b32, b2d, iden, z128, z512)
    return _combine(part, cnt)
